# baseline (speedup 1.0000x reference)
import functools

import jax
import jax.numpy as jnp
from jax import lax
from jax.experimental import pallas as pl
from jax.experimental.pallas import tpu as pltpu
from jax.experimental.pallas import tpu_sc as plsc

NC = 2
NS = 16
NW = NC * NS
LANES = 16
PACK = 4
LINE = 128
WAVE = 128


@functools.lru_cache(maxsize=None)
def _make_sc_kernel(B, D, b_per_w):
    assert D == 2 * LANES
    n_waves = b_per_w // WAVE
    mesh = plsc.VectorSubcoreMesh(core_axis_name="c", subcore_axis_name="s")

    @functools.partial(
        pl.kernel,
        out_type=jax.ShapeDtypeStruct((B,), jnp.float32),
        mesh=mesh,
        scratch_types=[
            pltpu.VMEM((b_per_w,), jnp.int32),
            pltpu.VMEM((b_per_w,), jnp.int32),
            pltpu.VMEM((WAVE, LINE), jnp.float32),
            pltpu.VMEM((WAVE, LINE), jnp.float32),
            pltpu.VMEM((b_per_w,), jnp.float32),
            pltpu.SemaphoreType.DMA,
            pltpu.SemaphoreType.DMA,
        ],
        compiler_params=pltpu.CompilerParams(needs_layout_passes=False),
    )
    def k(uids_hbm, iids_hbm, ut_hbm, it_hbm, out_hbm,
          uidx_v, iidx_v, ublk_v, iblk_v, out_v, sem_u, sem_i):
        wid = lax.axis_index("s") * NC + lax.axis_index("c")
        base = wid * b_per_w

        pltpu.sync_copy(uids_hbm.at[pl.ds(base, b_per_w)], uidx_v)
        pltpu.sync_copy(iids_hbm.at[pl.ds(base, b_per_w)], iidx_v)

        lane = lax.iota(jnp.int32, LANES)

        for w in range(n_waves):
            for c in range(WAVE // LANES):
                off = w * WAVE + c * LANES
                ugrp = lax.shift_right_logical(
                    uidx_v[pl.ds(off, LANES)], 2)
                igrp = lax.shift_right_logical(
                    iidx_v[pl.ds(off, LANES)], 2)
                pltpu.async_copy(
                    ut_hbm.at[ugrp], ublk_v.at[pl.ds(c * LANES, LANES)],
                    sem_u)
                pltpu.async_copy(
                    it_hbm.at[igrp], iblk_v.at[pl.ds(c * LANES, LANES)],
                    sem_i)
            pltpu.make_async_copy(
                ut_hbm.at[pl.ds(0, WAVE)], ublk_v, sem_u).wait()
            pltpu.make_async_copy(
                it_hbm.at[pl.ds(0, WAVE)], iblk_v, sem_i).wait()

            def group(g, carry):
                off = w * WAVE + g * LANES
                uvec = lax.bitwise_and(uidx_v[pl.ds(off, LANES)], PACK - 1)
                ivec = lax.bitwise_and(iidx_v[pl.ds(off, LANES)], PACK - 1)
                acc = jnp.zeros((LANES,), jnp.float32)
                for t in range(LANES):
                    r = g * LANES + t
                    uo = uvec[t] * D
                    io = ivec[t] * D
                    u0 = ublk_v[r, pl.ds(uo, LANES)]
                    u1 = ublk_v[r, pl.ds(uo + LANES, LANES)]
                    i0 = iblk_v[r, pl.ds(io, LANES)]
                    i1 = iblk_v[r, pl.ds(io + LANES, LANES)]
                    s = jnp.sum(u0 * i0 + u1 * i1)
                    acc = jnp.where(lane == t, s, acc)
                out_v[pl.ds(off, LANES)] = acc
                return carry
            lax.fori_loop(0, WAVE // LANES, group, 0)

        pltpu.sync_copy(out_v, out_hbm.at[pl.ds(base, b_per_w)])

    return k


def kernel(user_ids, item_ids, user_table, item_table):
    B = user_ids.shape[0]
    V, D = user_table.shape
    b_per_w = B // NW
    k = _make_sc_kernel(B, D, b_per_w)
    ut2 = user_table.reshape(V // PACK, LINE)
    it2 = item_table.reshape(V // PACK, LINE)
    return k(user_ids, item_ids, ut2, it2)
